# baseline (speedup 1.0000x reference)
import functools

import jax
import jax.numpy as jnp
from jax import lax
from jax.experimental import pallas as pl
from jax.experimental.pallas import tpu as pltpu
from jax.experimental.pallas import tpu_sc as plsc

N_NODES = 10000
EMBED = 128
HID = 64
N_EDGES = 320000
N_PAIRS = 16384

NC, NS = 2, 16
NW = NC * NS
EPW = N_EDGES // NW
CH = 125
NCH = EPW // CH
NPAD = 10240
RPT = NPAD // NS
PPW = N_PAIRS // NW
NB = 4
NBLK = 10
BLK = NPAD // NBLK


def _mesh():
    return plsc.VectorSubcoreMesh(core_axis_name="c", subcore_axis_name="s",
                                  num_cores=NC, num_subcores=NS)


_SC_PARAMS = pltpu.CompilerParams(use_tc_tiling_on_sc=False,
                                  needs_layout_passes=False)


@functools.cache
def _deg_kernel():
    @functools.partial(
        pl.kernel,
        out_type=jax.ShapeDtypeStruct((NC * NPAD, 16), jnp.float32),
        mesh=_mesh(),
        compiler_params=_SC_PARAMS,
        scratch_types=[
            pltpu.VMEM((NCH, CH), jnp.int32),
            pltpu.VMEM((CH, 16), jnp.float32),
            pltpu.VMEM((RPT, 16), jnp.float32),
            pltpu.VMEM_SHARED((NPAD, 16), jnp.float32),
        ],
    )
    def deg_kernel(e3_hbm, ones_hbm, zeros_hbm, out_hbm, dst_v, ones_v, zb_v,
                   bins_sh):
        c = lax.axis_index("c")
        s = lax.axis_index("s")
        wid = c * NS + s
        pltpu.sync_copy(e3_hbm.at[1, pl.ds(wid * NCH, NCH)], dst_v)
        pltpu.sync_copy(ones_hbm, ones_v)
        pltpu.sync_copy(zeros_hbm, zb_v)
        pltpu.sync_copy(zb_v, bins_sh.at[pl.ds(s * RPT, RPT)])
        plsc.subcore_barrier()

        def body(j, carry):
            pltpu.sync_copy(ones_v, bins_sh.at[dst_v.at[j]], add=True)
            return carry

        lax.fori_loop(0, NCH, body, 0)
        plsc.subcore_barrier()
        pltpu.sync_copy(bins_sh.at[pl.ds(s * RPT, RPT)], zb_v)
        pltpu.sync_copy(zb_v, out_hbm.at[pl.ds(c * NPAD + s * RPT, RPT)])

    return deg_kernel


@functools.cache
def _scatter_kernel():
    @functools.partial(
        pl.kernel,
        out_type=jax.ShapeDtypeStruct((NC * NPAD, HID), jnp.float32),
        mesh=_mesh(),
        compiler_params=_SC_PARAMS,
        scratch_types=[
            pltpu.VMEM((NCH, CH), jnp.int32),
            pltpu.VMEM((NCH, CH), jnp.int32),
            pltpu.VMEM((NB, CH, HID), jnp.float32),
            pltpu.VMEM((128, HID), jnp.float32),
            pltpu.VMEM_SHARED((NPAD, HID), jnp.float32),
            pltpu.SemaphoreType.DMA,
            pltpu.SemaphoreType.DMA,
        ],
    )
    def scatter_kernel(y_hbm, e3_hbm, zeros_hbm, out_hbm, src_v,
                       dst_v, rows_v, zb_v, acc_sh, gsem, ssem):
        c = lax.axis_index("c")
        s = lax.axis_index("s")
        wid = c * NS + s
        pltpu.sync_copy(e3_hbm.at[0, pl.ds(wid * NCH, NCH)], src_v)
        pltpu.sync_copy(e3_hbm.at[1, pl.ds(wid * NCH, NCH)], dst_v)
        pltpu.sync_copy(zeros_hbm, zb_v)
        for k in range(RPT // 128):
            pltpu.sync_copy(zb_v, acc_sh.at[pl.ds(s * RPT + k * 128, 128)])
        plsc.subcore_barrier()

        for b in range(2):
            pltpu.async_copy(y_hbm.at[src_v.at[b]], rows_v.at[b], gsem)

        def body(jo, carry):
            for b in range(NB):
                j = jo * NB + b
                pltpu.make_async_copy(
                    y_hbm.at[src_v.at[0]], rows_v.at[b], gsem).wait()
                pltpu.async_copy(rows_v.at[b], acc_sh.at[dst_v.at[j]], ssem,
                                 add=True)

                @pl.when(j >= 2)
                def _():
                    pltpu.make_async_copy(
                        rows_v.at[b], acc_sh.at[dst_v.at[0]], ssem).wait()

                @pl.when(j + 2 < NCH)
                def _():
                    pltpu.async_copy(y_hbm.at[src_v.at[j + 2]],
                                     rows_v.at[(b + 2) % NB], gsem)
            return carry

        lax.fori_loop(0, NCH // NB, body, 0)
        for _ in range(2):
            pltpu.make_async_copy(
                rows_v.at[0], acc_sh.at[dst_v.at[0]], ssem).wait()
        plsc.subcore_barrier()
        for k in range(RPT // 128):
            pltpu.sync_copy(acc_sh.at[pl.ds(s * RPT + k * 128, 128)], zb_v)
            pltpu.sync_copy(
                zb_v, out_hbm.at[pl.ds(c * NPAD + s * RPT + k * 128, 128)])

    return scatter_kernel


@functools.cache
def _pairs_kernel():
    @functools.partial(
        pl.kernel,
        out_type=jax.ShapeDtypeStruct((N_PAIRS, 2), jnp.float32),
        mesh=_mesh(),
        compiler_params=_SC_PARAMS,
        scratch_types=[
            pltpu.VMEM((NPAD, 4), jnp.float32),
            pltpu.VMEM((PPW,), jnp.int32),
            pltpu.VMEM((PPW,), jnp.int32),
            pltpu.VMEM((PPW, 2), jnp.float32),
        ],
    )
    def pairs_kernel(tab_hbm, n1_hbm, n2_hbm, out_hbm, tab_v, n1_v, n2_v, o_v):
        c = lax.axis_index("c")
        s = lax.axis_index("s")
        wid = c * NS + s
        pltpu.sync_copy(tab_hbm, tab_v)
        pltpu.sync_copy(n1_hbm.at[pl.ds(wid * PPW, PPW)], n1_v)
        pltpu.sync_copy(n2_hbm.at[pl.ds(wid * PPW, PPW)], n2_v)
        lanes = lax.iota(jnp.int32, 16)
        c0 = jnp.zeros((16,), jnp.int32)
        for i in range(PPW // 16):
            i1 = n1_v[pl.ds(i * 16, 16)]
            i2 = n2_v[pl.ds(i * 16, 16)]
            a0 = plsc.load_gather(tab_v, [i1, c0])
            a1 = plsc.load_gather(tab_v, [i1, c0 + 1])
            b0 = plsc.load_gather(tab_v, [i2, c0 + 2])
            b1 = plsc.load_gather(tab_v, [i2, c0 + 3])
            rows = i * 16 + lanes
            plsc.store_scatter(o_v, [rows, c0], a0 + b0)
            plsc.store_scatter(o_v, [rows, c0 + 1], a1 + b1)
        pltpu.sync_copy(o_v, out_hbm.at[pl.ds(wid * PPW, PPW)])

    return pairs_kernel


def _tc_mm_body(x_ref, w_ref, xw_ref):
    xw_ref[...] = jnp.dot(x_ref[...], w_ref[...],
                          preferred_element_type=jnp.float32)


def _tc_scale_body(xw_ref, d0_ref, d1_ref, b_ref, y_ref, w0_ref):
    xw = xw_ref[...]
    deg = d0_ref[...][:, 0] + d1_ref[...][:, 0] + 1.0
    dis = lax.rsqrt(deg)[:, None]
    y_ref[...] = dis * xw
    w0_ref[...] = dis * dis * xw + b_ref[...][0:1, :]


def _tc_post_body(t0_ref, t1_ref, w0_ref, d0_ref, d1_ref, wl_ref, bl_ref,
                  tab_ref):
    deg = d0_ref[...][:, 0] + d1_ref[...][:, 0] + 1.0
    dis = lax.rsqrt(deg)[:, None]
    h = jnp.maximum(dis * (t0_ref[...] + t1_ref[...]) + w0_ref[...], 0.0)
    tab_ref[...] = (
        jnp.dot(h, wl_ref[...], preferred_element_type=jnp.float32)
        + bl_ref[...][0:1, :])


def _tc_mm(x, W_gcn):
    return pl.pallas_call(
        _tc_mm_body,
        grid=(NBLK,),
        in_specs=[
            pl.BlockSpec((BLK, EMBED), lambda i: (i, 0)),
            pl.BlockSpec((EMBED, HID), lambda i: (0, 0)),
        ],
        out_specs=pl.BlockSpec((BLK, HID), lambda i: (i, 0)),
        out_shape=jax.ShapeDtypeStruct((NPAD, HID), jnp.float32),
    )(x, W_gcn)


def _tc_scale(xw, deg_part, b_gcn):
    return pl.pallas_call(
        _tc_scale_body,
        grid=(NBLK,),
        in_specs=[
            pl.BlockSpec((BLK, HID), lambda i: (i, 0)),
            pl.BlockSpec((BLK, 16), lambda i: (i, 0)),
            pl.BlockSpec((BLK, 16), lambda i: (i + NBLK, 0)),
            pl.BlockSpec((8, HID), lambda i: (0, 0)),
        ],
        out_specs=[
            pl.BlockSpec((BLK, HID), lambda i: (i, 0)),
            pl.BlockSpec((BLK, HID), lambda i: (i, 0)),
        ],
        out_shape=[jax.ShapeDtypeStruct((NPAD, HID), jnp.float32)] * 2,
    )(xw, deg_part, deg_part,
      jnp.broadcast_to(b_gcn[None, :], (8, HID)))


def _tc_post(t_part, w0, deg_part, wl4, bl4):
    return pl.pallas_call(
        _tc_post_body,
        grid=(NBLK,),
        in_specs=[
            pl.BlockSpec((BLK, HID), lambda i: (i, 0)),
            pl.BlockSpec((BLK, HID), lambda i: (i + NBLK, 0)),
            pl.BlockSpec((BLK, HID), lambda i: (i, 0)),
            pl.BlockSpec((BLK, 16), lambda i: (i, 0)),
            pl.BlockSpec((BLK, 16), lambda i: (i + NBLK, 0)),
            pl.BlockSpec((HID, 4), lambda i: (0, 0)),
            pl.BlockSpec((8, 4), lambda i: (0, 0)),
        ],
        out_specs=pl.BlockSpec((BLK, 4), lambda i: (i, 0)),
        out_shape=jax.ShapeDtypeStruct((NPAD, 4), jnp.float32),
    )(t_part, t_part, w0, deg_part, deg_part, wl4, bl4)


def kernel(x, edge_index, node1, node2, W_gcn, b_gcn, W_lin, b_lin):
    x = jnp.pad(x, ((0, NPAD - N_NODES), (0, 0)))
    e3 = edge_index.reshape(2, NW * NCH, CH)
    ones16 = jnp.ones((CH, 16), jnp.float32)
    zeros16 = jnp.zeros((RPT, 16), jnp.float32)
    zeros64 = jnp.zeros((128, HID), jnp.float32)

    deg_part = _deg_kernel()(e3, ones16, zeros16)
    xw = _tc_mm(x, W_gcn)
    y, w0 = _tc_scale(xw, deg_part, b_gcn)
    t_part = _scatter_kernel()(y, e3, zeros64)

    wl4 = jnp.concatenate([W_lin[:HID], W_lin[HID:]], axis=1)
    bl4 = jnp.broadcast_to(
        jnp.concatenate([b_lin, jnp.zeros((2,), jnp.float32)])[None, :], (8, 4))
    tab = _tc_post(t_part, w0, deg_part, wl4, bl4)

    return _pairs_kernel()(tab, node1, node2)


# speedup vs baseline: 1.0444x; 1.0444x over previous
import functools

import jax
import jax.numpy as jnp
from jax import lax
from jax.experimental import pallas as pl
from jax.experimental.pallas import tpu as pltpu
from jax.experimental.pallas import tpu_sc as plsc

N_NODES = 10000
EMBED = 128
HID = 64
N_EDGES = 320000
N_PAIRS = 16384

NC, NS = 2, 16
NW = NC * NS
EPW = N_EDGES // NW
CH = 125
NCH = EPW // CH
NPAD = 10240
RPT = NPAD // NS
PPW = N_PAIRS // NW
NB = 4
NBLK = 10
BLK = NPAD // NBLK


def _mesh():
    return plsc.VectorSubcoreMesh(core_axis_name="c", subcore_axis_name="s",
                                  num_cores=NC, num_subcores=NS)


_SC_PARAMS = pltpu.CompilerParams(use_tc_tiling_on_sc=False,
                                  needs_layout_passes=False)


@functools.cache
def _deg_kernel():
    @functools.partial(
        pl.kernel,
        out_type=jax.ShapeDtypeStruct((NC * NPAD, 16), jnp.float32),
        mesh=_mesh(),
        compiler_params=_SC_PARAMS,
        scratch_types=[
            pltpu.VMEM((NCH, CH), jnp.int32),
            pltpu.VMEM((CH, 16), jnp.float32),
            pltpu.VMEM((RPT, 16), jnp.float32),
            pltpu.VMEM_SHARED((NPAD, 16), jnp.float32),
        ],
    )
    def deg_kernel(e3_hbm, ones_hbm, zeros_hbm, out_hbm, dst_v, ones_v, zb_v,
                   bins_sh):
        c = lax.axis_index("c")
        s = lax.axis_index("s")
        wid = c * NS + s
        pltpu.sync_copy(e3_hbm.at[1, pl.ds(wid * NCH, NCH)], dst_v)
        pltpu.sync_copy(ones_hbm, ones_v)
        pltpu.sync_copy(zeros_hbm, zb_v)
        pltpu.sync_copy(zb_v, bins_sh.at[pl.ds(s * RPT, RPT)])
        plsc.subcore_barrier()

        def body(j, carry):
            pltpu.sync_copy(ones_v, bins_sh.at[dst_v.at[j]], add=True)
            return carry

        lax.fori_loop(0, NCH, body, 0)
        plsc.subcore_barrier()
        pltpu.sync_copy(bins_sh.at[pl.ds(s * RPT, RPT)], zb_v)
        pltpu.sync_copy(zb_v, out_hbm.at[pl.ds(c * NPAD + s * RPT, RPT)])

    return deg_kernel


@functools.cache
def _scatter_kernel():
    @functools.partial(
        pl.kernel,
        out_type=jax.ShapeDtypeStruct((NC * NPAD, HID), jnp.float32),
        mesh=_mesh(),
        compiler_params=_SC_PARAMS,
        scratch_types=[
            pltpu.VMEM((NCH, CH), jnp.int32),
            pltpu.VMEM((NCH, CH), jnp.int32),
            pltpu.VMEM((NB, CH, HID), jnp.float32),
            pltpu.VMEM((2, 128, HID), jnp.float32),
            pltpu.VMEM_SHARED((NPAD, HID), jnp.float32),
            pltpu.SemaphoreType.DMA,
        ],
    )
    def scatter_kernel(y_hbm, e3_hbm, zeros_hbm, out_hbm, src_v,
                       dst_v, rows_v, zb_v, acc_sh, gsem):
        c = lax.axis_index("c")
        s = lax.axis_index("s")
        wid = c * NS + s
        pltpu.sync_copy(e3_hbm.at[0, pl.ds(wid * NCH, NCH)], src_v)
        pltpu.sync_copy(e3_hbm.at[1, pl.ds(wid * NCH, NCH)], dst_v)
        pltpu.sync_copy(zeros_hbm, zb_v.at[0])
        for k in range(RPT // 128):
            pltpu.async_copy(
                zb_v.at[0], acc_sh.at[pl.ds(s * RPT + k * 128, 128)], gsem)
        for k in range(RPT // 128):
            pltpu.make_async_copy(
                zb_v.at[0], acc_sh.at[pl.ds(s * RPT, 128)], gsem).wait()
        plsc.subcore_barrier()

        for b in range(NB):
            pltpu.async_copy(y_hbm.at[src_v.at[b]], rows_v.at[b], gsem)

        def body(jo, carry):
            for b in range(NB):
                j = jo * NB + b
                pltpu.make_async_copy(
                    y_hbm.at[src_v.at[0]], rows_v.at[b], gsem).wait()
                pltpu.sync_copy(rows_v.at[b], acc_sh.at[dst_v.at[j]],
                                add=True)

                @pl.when(jo < NCH // NB - 1)
                def _():
                    pltpu.async_copy(y_hbm.at[src_v.at[j + NB]], rows_v.at[b],
                                     gsem)
            return carry

        lax.fori_loop(0, NCH // NB, body, 0)
        plsc.subcore_barrier()
        for k in range(RPT // 128):
            b = k % 2
            pltpu.sync_copy(acc_sh.at[pl.ds(s * RPT + k * 128, 128)],
                            zb_v.at[b])
            pltpu.async_copy(
                zb_v.at[b],
                out_hbm.at[pl.ds(c * NPAD + s * RPT + k * 128, 128)], gsem)
            if k >= 1:
                pltpu.make_async_copy(
                    zb_v.at[b],
                    out_hbm.at[pl.ds(c * NPAD, 128)], gsem).wait()
        pltpu.make_async_copy(
            zb_v.at[0], out_hbm.at[pl.ds(c * NPAD, 128)], gsem).wait()

    return scatter_kernel


@functools.cache
def _pairs_kernel():
    @functools.partial(
        pl.kernel,
        out_type=jax.ShapeDtypeStruct((N_PAIRS, 2), jnp.float32),
        mesh=_mesh(),
        compiler_params=_SC_PARAMS,
        scratch_types=[
            pltpu.VMEM((NPAD, 4), jnp.float32),
            pltpu.VMEM((PPW,), jnp.int32),
            pltpu.VMEM((PPW,), jnp.int32),
            pltpu.VMEM((PPW, 2), jnp.float32),
        ],
    )
    def pairs_kernel(tab_hbm, n1_hbm, n2_hbm, out_hbm, tab_v, n1_v, n2_v, o_v):
        c = lax.axis_index("c")
        s = lax.axis_index("s")
        wid = c * NS + s
        pltpu.sync_copy(tab_hbm, tab_v)
        pltpu.sync_copy(n1_hbm.at[pl.ds(wid * PPW, PPW)], n1_v)
        pltpu.sync_copy(n2_hbm.at[pl.ds(wid * PPW, PPW)], n2_v)
        lanes = lax.iota(jnp.int32, 16)
        c0 = jnp.zeros((16,), jnp.int32)
        for i in range(PPW // 16):
            i1 = n1_v[pl.ds(i * 16, 16)]
            i2 = n2_v[pl.ds(i * 16, 16)]
            a0 = plsc.load_gather(tab_v, [i1, c0])
            a1 = plsc.load_gather(tab_v, [i1, c0 + 1])
            b0 = plsc.load_gather(tab_v, [i2, c0 + 2])
            b1 = plsc.load_gather(tab_v, [i2, c0 + 3])
            rows = i * 16 + lanes
            plsc.store_scatter(o_v, [rows, c0], a0 + b0)
            plsc.store_scatter(o_v, [rows, c0 + 1], a1 + b1)
        pltpu.sync_copy(o_v, out_hbm.at[pl.ds(wid * PPW, PPW)])

    return pairs_kernel


def _tc_mm_body(x_ref, w_ref, xw_ref):
    xw_ref[...] = jnp.dot(x_ref[...], w_ref[...],
                          preferred_element_type=jnp.float32)


def _tc_scale_body(xw_ref, d0_ref, d1_ref, b_ref, y_ref, w0_ref):
    xw = xw_ref[...]
    deg = d0_ref[...][:, 0] + d1_ref[...][:, 0] + 1.0
    dis = lax.rsqrt(deg)[:, None]
    y_ref[...] = dis * xw
    w0_ref[...] = dis * dis * xw + b_ref[...][0:1, :]


def _tc_post_body(t0_ref, t1_ref, w0_ref, d0_ref, d1_ref, wl_ref, bl_ref,
                  tab_ref):
    deg = d0_ref[...][:, 0] + d1_ref[...][:, 0] + 1.0
    dis = lax.rsqrt(deg)[:, None]
    h = jnp.maximum(dis * (t0_ref[...] + t1_ref[...]) + w0_ref[...], 0.0)
    tab_ref[...] = (
        jnp.dot(h, wl_ref[...], preferred_element_type=jnp.float32)
        + bl_ref[...][0:1, :])


def _tc_mm(x, W_gcn):
    return pl.pallas_call(
        _tc_mm_body,
        grid=(NBLK,),
        in_specs=[
            pl.BlockSpec((BLK, EMBED), lambda i: (i, 0)),
            pl.BlockSpec((EMBED, HID), lambda i: (0, 0)),
        ],
        out_specs=pl.BlockSpec((BLK, HID), lambda i: (i, 0)),
        out_shape=jax.ShapeDtypeStruct((NPAD, HID), jnp.float32),
    )(x, W_gcn)


def _tc_scale(xw, deg_part, b_gcn):
    return pl.pallas_call(
        _tc_scale_body,
        grid=(NBLK,),
        in_specs=[
            pl.BlockSpec((BLK, HID), lambda i: (i, 0)),
            pl.BlockSpec((BLK, 16), lambda i: (i, 0)),
            pl.BlockSpec((BLK, 16), lambda i: (i + NBLK, 0)),
            pl.BlockSpec((8, HID), lambda i: (0, 0)),
        ],
        out_specs=[
            pl.BlockSpec((BLK, HID), lambda i: (i, 0)),
            pl.BlockSpec((BLK, HID), lambda i: (i, 0)),
        ],
        out_shape=[jax.ShapeDtypeStruct((NPAD, HID), jnp.float32)] * 2,
    )(xw, deg_part, deg_part,
      jnp.broadcast_to(b_gcn[None, :], (8, HID)))


def _tc_post(t_part, w0, deg_part, wl4, bl4):
    return pl.pallas_call(
        _tc_post_body,
        grid=(NBLK,),
        in_specs=[
            pl.BlockSpec((BLK, HID), lambda i: (i, 0)),
            pl.BlockSpec((BLK, HID), lambda i: (i + NBLK, 0)),
            pl.BlockSpec((BLK, HID), lambda i: (i, 0)),
            pl.BlockSpec((BLK, 16), lambda i: (i, 0)),
            pl.BlockSpec((BLK, 16), lambda i: (i + NBLK, 0)),
            pl.BlockSpec((HID, 4), lambda i: (0, 0)),
            pl.BlockSpec((8, 4), lambda i: (0, 0)),
        ],
        out_specs=pl.BlockSpec((BLK, 4), lambda i: (i, 0)),
        out_shape=jax.ShapeDtypeStruct((NPAD, 4), jnp.float32),
    )(t_part, t_part, w0, deg_part, deg_part, wl4, bl4)


def kernel(x, edge_index, node1, node2, W_gcn, b_gcn, W_lin, b_lin):
    x = jnp.pad(x, ((0, NPAD - N_NODES), (0, 0)))
    e3 = edge_index.reshape(2, NW * NCH, CH)
    ones16 = jnp.ones((CH, 16), jnp.float32)
    zeros16 = jnp.zeros((RPT, 16), jnp.float32)
    zeros64 = jnp.zeros((128, HID), jnp.float32)

    deg_part = _deg_kernel()(e3, ones16, zeros16)
    xw = _tc_mm(x, W_gcn)
    y, w0 = _tc_scale(xw, deg_part, b_gcn)
    t_part = _scatter_kernel()(y, e3, zeros64)

    wl4 = jnp.concatenate([W_lin[:HID], W_lin[HID:]], axis=1)
    bl4 = jnp.broadcast_to(
        jnp.concatenate([b_lin, jnp.zeros((2,), jnp.float32)])[None, :], (8, 4))
    tab = _tc_post(t_part, w0, deg_part, wl4, bl4)

    return _pairs_kernel()(tab, node1, node2)


# speedup vs baseline: 1.0638x; 1.0186x over previous
import functools

import jax
import jax.numpy as jnp
from jax import lax
from jax.experimental import pallas as pl
from jax.experimental.pallas import tpu as pltpu
from jax.experimental.pallas import tpu_sc as plsc

N_NODES = 10000
EMBED = 128
HID = 64
N_EDGES = 320000
N_PAIRS = 16384

NC, NS = 2, 16
NW = NC * NS
EPW = N_EDGES // NW
CH = 125
NCH = EPW // CH
NPAD = 10240
RPT = NPAD // NS
PPW = N_PAIRS // NW
NB = 4
NBLK = 5
BLK = NPAD // NBLK


def _mesh():
    return plsc.VectorSubcoreMesh(core_axis_name="c", subcore_axis_name="s",
                                  num_cores=NC, num_subcores=NS)


_SC_PARAMS = pltpu.CompilerParams(use_tc_tiling_on_sc=False,
                                  needs_layout_passes=False)


@functools.cache
def _deg_kernel():
    @functools.partial(
        pl.kernel,
        out_type=jax.ShapeDtypeStruct((NC * NPAD, 16), jnp.float32),
        mesh=_mesh(),
        compiler_params=_SC_PARAMS,
        scratch_types=[
            pltpu.VMEM((NCH, CH), jnp.int32),
            pltpu.VMEM((CH, 16), jnp.float32),
            pltpu.VMEM((RPT, 16), jnp.float32),
            pltpu.VMEM_SHARED((NPAD, 16), jnp.float32),
        ],
    )
    def deg_kernel(e3_hbm, ones_hbm, zeros_hbm, out_hbm, dst_v, ones_v, zb_v,
                   bins_sh):
        c = lax.axis_index("c")
        s = lax.axis_index("s")
        wid = c * NS + s
        pltpu.sync_copy(e3_hbm.at[1, pl.ds(wid * NCH, NCH)], dst_v)
        pltpu.sync_copy(ones_hbm, ones_v)
        pltpu.sync_copy(zeros_hbm, zb_v)
        pltpu.sync_copy(zb_v, bins_sh.at[pl.ds(s * RPT, RPT)])
        plsc.subcore_barrier()


        def body(j, carry):
            pltpu.sync_copy(ones_v, bins_sh.at[dst_v.at[j]], add=True)
            return carry

        lax.fori_loop(0, NCH, body, 0)
        plsc.subcore_barrier()
        pltpu.sync_copy(bins_sh.at[pl.ds(s * RPT, RPT)], zb_v)
        pltpu.sync_copy(zb_v, out_hbm.at[pl.ds(c * NPAD + s * RPT, RPT)])

    return deg_kernel


@functools.cache
def _scatter_kernel():
    @functools.partial(
        pl.kernel,
        out_type=jax.ShapeDtypeStruct((NC * NPAD, HID), jnp.float32),
        mesh=_mesh(),
        compiler_params=_SC_PARAMS,
        scratch_types=[
            pltpu.VMEM((NCH, CH), jnp.int32),
            pltpu.VMEM((NCH, CH), jnp.int32),
            pltpu.VMEM((NB, CH, HID), jnp.float32),
            pltpu.VMEM((2, 128, HID), jnp.float32),
            pltpu.VMEM_SHARED((NPAD, HID), jnp.float32),
            pltpu.SemaphoreType.DMA,
        ],
    )
    def scatter_kernel(y_hbm, e3_hbm, zeros_hbm, out_hbm, src_v,
                       dst_v, rows_v, zb_v, acc_sh, gsem):
        c = lax.axis_index("c")
        s = lax.axis_index("s")
        wid = c * NS + s
        pltpu.sync_copy(e3_hbm.at[0, pl.ds(wid * NCH, NCH)], src_v)
        pltpu.sync_copy(e3_hbm.at[1, pl.ds(wid * NCH, NCH)], dst_v)
        pltpu.sync_copy(zeros_hbm, zb_v.at[0])
        for k in range(RPT // 128):
            pltpu.async_copy(
                zb_v.at[0], acc_sh.at[pl.ds(s * RPT + k * 128, 128)], gsem)
        for k in range(RPT // 128):
            pltpu.make_async_copy(
                zb_v.at[0], acc_sh.at[pl.ds(s * RPT, 128)], gsem).wait()
        plsc.subcore_barrier()

        for b in range(NB):
            pltpu.async_copy(y_hbm.at[src_v.at[b]], rows_v.at[b], gsem)

        def body(jo, carry):
            for b in range(NB):
                j = jo * NB + b
                pltpu.make_async_copy(
                    y_hbm.at[src_v.at[0]], rows_v.at[b], gsem).wait()
                pltpu.sync_copy(rows_v.at[b], acc_sh.at[dst_v.at[j]],
                                add=True)

                @pl.when(jo < NCH // NB - 1)
                def _():
                    pltpu.async_copy(y_hbm.at[src_v.at[j + NB]], rows_v.at[b],
                                     gsem)
            return carry

        lax.fori_loop(0, NCH // NB, body, 0)
        plsc.subcore_barrier()
        for k in range(RPT // 128):
            b = k % 2
            pltpu.sync_copy(acc_sh.at[pl.ds(s * RPT + k * 128, 128)],
                            zb_v.at[b])
            pltpu.async_copy(
                zb_v.at[b],
                out_hbm.at[pl.ds(c * NPAD + s * RPT + k * 128, 128)], gsem)
            if k >= 1:
                pltpu.make_async_copy(
                    zb_v.at[b],
                    out_hbm.at[pl.ds(c * NPAD, 128)], gsem).wait()
        pltpu.make_async_copy(
            zb_v.at[0], out_hbm.at[pl.ds(c * NPAD, 128)], gsem).wait()

    return scatter_kernel


@functools.cache
def _pairs_kernel():
    @functools.partial(
        pl.kernel,
        out_type=jax.ShapeDtypeStruct((N_PAIRS, 2), jnp.float32),
        mesh=_mesh(),
        compiler_params=_SC_PARAMS,
        scratch_types=[
            pltpu.VMEM((NPAD, 4), jnp.float32),
            pltpu.VMEM((PPW,), jnp.int32),
            pltpu.VMEM((PPW,), jnp.int32),
            pltpu.VMEM((PPW, 2), jnp.float32),
        ],
    )
    def pairs_kernel(tab_hbm, n1_hbm, n2_hbm, out_hbm, tab_v, n1_v, n2_v, o_v):
        c = lax.axis_index("c")
        s = lax.axis_index("s")
        wid = c * NS + s
        pltpu.sync_copy(tab_hbm, tab_v)
        pltpu.sync_copy(n1_hbm.at[pl.ds(wid * PPW, PPW)], n1_v)
        pltpu.sync_copy(n2_hbm.at[pl.ds(wid * PPW, PPW)], n2_v)
        lanes = lax.iota(jnp.int32, 16)
        c0 = jnp.zeros((16,), jnp.int32)
        for i in range(PPW // 16):
            i1 = n1_v[pl.ds(i * 16, 16)]
            i2 = n2_v[pl.ds(i * 16, 16)]
            a0 = plsc.load_gather(tab_v, [i1, c0])
            a1 = plsc.load_gather(tab_v, [i1, c0 + 1])
            b0 = plsc.load_gather(tab_v, [i2, c0 + 2])
            b1 = plsc.load_gather(tab_v, [i2, c0 + 3])
            rows = i * 16 + lanes
            plsc.store_scatter(o_v, [rows, c0], a0 + b0)
            plsc.store_scatter(o_v, [rows, c0 + 1], a1 + b1)
        pltpu.sync_copy(o_v, out_hbm.at[pl.ds(wid * PPW, PPW)])

    return pairs_kernel


def _tc_mm_body(x_ref, w_ref, xw_ref):
    xw_ref[...] = jnp.dot(x_ref[...], w_ref[...],
                          preferred_element_type=jnp.float32)


def _tc_scale_body(xw_ref, d0_ref, d1_ref, b_ref, y_ref, w0_ref):
    xw = xw_ref[...]
    deg = d0_ref[...][:, 0] + d1_ref[...][:, 0] + 1.0
    dis = lax.rsqrt(deg)[:, None]
    y_ref[...] = dis * xw
    w0_ref[...] = dis * dis * xw + b_ref[...][0:1, :]


def _tc_post_body(t0_ref, t1_ref, w0_ref, d0_ref, d1_ref, wl_ref, bl_ref,
                  tab_ref):
    deg = d0_ref[...][:, 0] + d1_ref[...][:, 0] + 1.0
    dis = lax.rsqrt(deg)[:, None]
    h = jnp.maximum(dis * (t0_ref[...] + t1_ref[...]) + w0_ref[...], 0.0)
    tab_ref[...] = (
        jnp.dot(h, wl_ref[...], preferred_element_type=jnp.float32)
        + bl_ref[...][0:1, :])


def _tc_mm(x, W_gcn):
    return pl.pallas_call(
        _tc_mm_body,
        grid=(NBLK,),
        in_specs=[
            pl.BlockSpec((BLK, EMBED), lambda i: (i, 0)),
            pl.BlockSpec((EMBED, HID), lambda i: (0, 0)),
        ],
        out_specs=pl.BlockSpec((BLK, HID), lambda i: (i, 0)),
        out_shape=jax.ShapeDtypeStruct((NPAD, HID), jnp.float32),
    )(x, W_gcn)


def _tc_scale(xw, deg_part, b_gcn):
    return pl.pallas_call(
        _tc_scale_body,
        grid=(NBLK,),
        in_specs=[
            pl.BlockSpec((BLK, HID), lambda i: (i, 0)),
            pl.BlockSpec((BLK, 16), lambda i: (i, 0)),
            pl.BlockSpec((BLK, 16), lambda i: (i + NBLK, 0)),
            pl.BlockSpec((8, HID), lambda i: (0, 0)),
        ],
        out_specs=[
            pl.BlockSpec((BLK, HID), lambda i: (i, 0)),
            pl.BlockSpec((BLK, HID), lambda i: (i, 0)),
        ],
        out_shape=[jax.ShapeDtypeStruct((NPAD, HID), jnp.float32)] * 2,
    )(xw, deg_part, deg_part,
      jnp.broadcast_to(b_gcn[None, :], (8, HID)))


def _tc_post(t_part, w0, deg_part, wl4, bl4):
    return pl.pallas_call(
        _tc_post_body,
        grid=(NBLK,),
        in_specs=[
            pl.BlockSpec((BLK, HID), lambda i: (i, 0)),
            pl.BlockSpec((BLK, HID), lambda i: (i + NBLK, 0)),
            pl.BlockSpec((BLK, HID), lambda i: (i, 0)),
            pl.BlockSpec((BLK, 16), lambda i: (i, 0)),
            pl.BlockSpec((BLK, 16), lambda i: (i + NBLK, 0)),
            pl.BlockSpec((HID, 4), lambda i: (0, 0)),
            pl.BlockSpec((8, 4), lambda i: (0, 0)),
        ],
        out_specs=pl.BlockSpec((BLK, 4), lambda i: (i, 0)),
        out_shape=jax.ShapeDtypeStruct((NPAD, 4), jnp.float32),
    )(t_part, t_part, w0, deg_part, deg_part, wl4, bl4)


def kernel(x, edge_index, node1, node2, W_gcn, b_gcn, W_lin, b_lin):
    x = jnp.pad(x, ((0, NPAD - N_NODES), (0, 0)))
    e3 = edge_index.reshape(2, NW * NCH, CH)
    ones16 = jnp.ones((CH, 16), jnp.float32)
    zeros16 = jnp.zeros((RPT, 16), jnp.float32)
    zeros64 = jnp.zeros((128, HID), jnp.float32)

    deg_part = _deg_kernel()(e3, ones16, zeros16)
    xw = _tc_mm(x, W_gcn)
    y, w0 = _tc_scale(xw, deg_part, b_gcn)
    t_part = _scatter_kernel()(y, e3, zeros64)

    wl4 = jnp.concatenate([W_lin[:HID], W_lin[HID:]], axis=1)
    bl4 = jnp.broadcast_to(
        jnp.concatenate([b_lin, jnp.zeros((2,), jnp.float32)])[None, :], (8, 4))
    tab = _tc_post(t_part, w0, deg_part, wl4, bl4)

    return _pairs_kernel()(tab, node1, node2)


# speedup vs baseline: 1.0763x; 1.0117x over previous
import functools

import jax
import jax.numpy as jnp
from jax import lax
from jax.experimental import pallas as pl
from jax.experimental.pallas import tpu as pltpu
from jax.experimental.pallas import tpu_sc as plsc

N_NODES = 10000
EMBED = 128
HID = 64
N_EDGES = 320000
N_PAIRS = 16384

NC, NS = 2, 16
NW = NC * NS
EPW = N_EDGES // NW
CH = 125
NCH = EPW // CH
NPAD = 10240
RPT = NPAD // NS
PPW = N_PAIRS // NW
NB = 4
NBLK = 2
BLK = NPAD // NBLK


def _mesh():
    return plsc.VectorSubcoreMesh(core_axis_name="c", subcore_axis_name="s",
                                  num_cores=NC, num_subcores=NS)


_SC_PARAMS = pltpu.CompilerParams(use_tc_tiling_on_sc=False,
                                  needs_layout_passes=False)


@functools.cache
def _deg_kernel():
    @functools.partial(
        pl.kernel,
        out_type=jax.ShapeDtypeStruct((NC * NPAD, 16), jnp.float32),
        mesh=_mesh(),
        compiler_params=_SC_PARAMS,
        scratch_types=[
            pltpu.VMEM((NCH, CH), jnp.int32),
            pltpu.VMEM((CH, 16), jnp.float32),
            pltpu.VMEM((RPT, 16), jnp.float32),
            pltpu.VMEM_SHARED((NPAD, 16), jnp.float32),
        ],
    )
    def deg_kernel(e3_hbm, ones_hbm, zeros_hbm, out_hbm, dst_v, ones_v, zb_v,
                   bins_sh):
        c = lax.axis_index("c")
        s = lax.axis_index("s")
        wid = c * NS + s
        pltpu.sync_copy(e3_hbm.at[1, pl.ds(wid * NCH, NCH)], dst_v)
        pltpu.sync_copy(ones_hbm, ones_v)
        pltpu.sync_copy(zeros_hbm, zb_v)
        pltpu.sync_copy(zb_v, bins_sh.at[pl.ds(s * RPT, RPT)])
        plsc.subcore_barrier()


        def body(j, carry):
            pltpu.sync_copy(ones_v, bins_sh.at[dst_v.at[j]], add=True)
            return carry

        lax.fori_loop(0, NCH, body, 0)
        plsc.subcore_barrier()
        pltpu.sync_copy(bins_sh.at[pl.ds(s * RPT, RPT)], zb_v)
        pltpu.sync_copy(zb_v, out_hbm.at[pl.ds(c * NPAD + s * RPT, RPT)])

    return deg_kernel


@functools.cache
def _scatter_kernel():
    @functools.partial(
        pl.kernel,
        out_type=jax.ShapeDtypeStruct((NC * NPAD, HID), jnp.float32),
        mesh=_mesh(),
        compiler_params=_SC_PARAMS,
        scratch_types=[
            pltpu.VMEM((NCH, CH), jnp.int32),
            pltpu.VMEM((NCH, CH), jnp.int32),
            pltpu.VMEM((NB, CH, HID), jnp.float32),
            pltpu.VMEM((2, 128, HID), jnp.float32),
            pltpu.VMEM_SHARED((NPAD, HID), jnp.float32),
            pltpu.SemaphoreType.DMA,
        ],
    )
    def scatter_kernel(y_hbm, e3_hbm, zeros_hbm, out_hbm, src_v,
                       dst_v, rows_v, zb_v, acc_sh, gsem):
        c = lax.axis_index("c")
        s = lax.axis_index("s")
        wid = c * NS + s
        pltpu.sync_copy(e3_hbm.at[0, pl.ds(wid * NCH, NCH)], src_v)
        pltpu.sync_copy(e3_hbm.at[1, pl.ds(wid * NCH, NCH)], dst_v)
        pltpu.sync_copy(zeros_hbm, zb_v.at[0])
        for k in range(RPT // 128):
            pltpu.async_copy(
                zb_v.at[0], acc_sh.at[pl.ds(s * RPT + k * 128, 128)], gsem)
        for k in range(RPT // 128):
            pltpu.make_async_copy(
                zb_v.at[0], acc_sh.at[pl.ds(s * RPT, 128)], gsem).wait()
        plsc.subcore_barrier()

        for b in range(NB):
            pltpu.async_copy(y_hbm.at[src_v.at[b]], rows_v.at[b], gsem)

        def body(jo, carry):
            for b in range(NB):
                j = jo * NB + b
                pltpu.make_async_copy(
                    y_hbm.at[src_v.at[0]], rows_v.at[b], gsem).wait()
                pltpu.sync_copy(rows_v.at[b], acc_sh.at[dst_v.at[j]],
                                add=True)

                @pl.when(jo < NCH // NB - 1)
                def _():
                    pltpu.async_copy(y_hbm.at[src_v.at[j + NB]], rows_v.at[b],
                                     gsem)
            return carry

        lax.fori_loop(0, NCH // NB, body, 0)
        plsc.subcore_barrier()
        for k in range(RPT // 128):
            b = k % 2
            pltpu.sync_copy(acc_sh.at[pl.ds(s * RPT + k * 128, 128)],
                            zb_v.at[b])
            pltpu.async_copy(
                zb_v.at[b],
                out_hbm.at[pl.ds(c * NPAD + s * RPT + k * 128, 128)], gsem)
            if k >= 1:
                pltpu.make_async_copy(
                    zb_v.at[b],
                    out_hbm.at[pl.ds(c * NPAD, 128)], gsem).wait()
        pltpu.make_async_copy(
            zb_v.at[0], out_hbm.at[pl.ds(c * NPAD, 128)], gsem).wait()

    return scatter_kernel


@functools.cache
def _pairs_kernel():
    @functools.partial(
        pl.kernel,
        out_type=jax.ShapeDtypeStruct((N_PAIRS, 2), jnp.float32),
        mesh=_mesh(),
        compiler_params=_SC_PARAMS,
        scratch_types=[
            pltpu.VMEM((NPAD, 4), jnp.float32),
            pltpu.VMEM((PPW,), jnp.int32),
            pltpu.VMEM((PPW,), jnp.int32),
            pltpu.VMEM((PPW, 2), jnp.float32),
        ],
    )
    def pairs_kernel(tab_hbm, n1_hbm, n2_hbm, out_hbm, tab_v, n1_v, n2_v, o_v):
        c = lax.axis_index("c")
        s = lax.axis_index("s")
        wid = c * NS + s
        pltpu.sync_copy(tab_hbm, tab_v)
        pltpu.sync_copy(n1_hbm.at[pl.ds(wid * PPW, PPW)], n1_v)
        pltpu.sync_copy(n2_hbm.at[pl.ds(wid * PPW, PPW)], n2_v)
        lanes = lax.iota(jnp.int32, 16)
        c0 = jnp.zeros((16,), jnp.int32)
        for i in range(PPW // 16):
            i1 = n1_v[pl.ds(i * 16, 16)]
            i2 = n2_v[pl.ds(i * 16, 16)]
            a0 = plsc.load_gather(tab_v, [i1, c0])
            a1 = plsc.load_gather(tab_v, [i1, c0 + 1])
            b0 = plsc.load_gather(tab_v, [i2, c0 + 2])
            b1 = plsc.load_gather(tab_v, [i2, c0 + 3])
            rows = i * 16 + lanes
            plsc.store_scatter(o_v, [rows, c0], a0 + b0)
            plsc.store_scatter(o_v, [rows, c0 + 1], a1 + b1)
        pltpu.sync_copy(o_v, out_hbm.at[pl.ds(wid * PPW, PPW)])

    return pairs_kernel


def _tc_mm_body(x_ref, w_ref, xw_ref):
    xw_ref[...] = jnp.dot(x_ref[...], w_ref[...],
                          preferred_element_type=jnp.float32)


def _tc_scale_body(xw_ref, d0_ref, d1_ref, b_ref, y_ref, w0_ref):
    xw = xw_ref[...]
    deg = d0_ref[...][:, 0] + d1_ref[...][:, 0] + 1.0
    dis = lax.rsqrt(deg)[:, None]
    y_ref[...] = dis * xw
    w0_ref[...] = dis * dis * xw + b_ref[...][0:1, :]


def _tc_post_body(t0_ref, t1_ref, w0_ref, d0_ref, d1_ref, wl_ref, bl_ref,
                  tab_ref):
    deg = d0_ref[...][:, 0] + d1_ref[...][:, 0] + 1.0
    dis = lax.rsqrt(deg)[:, None]
    h = jnp.maximum(dis * (t0_ref[...] + t1_ref[...]) + w0_ref[...], 0.0)
    tab_ref[...] = (
        jnp.dot(h, wl_ref[...], preferred_element_type=jnp.float32)
        + bl_ref[...][0:1, :])


def _tc_mm(x, W_gcn):
    return pl.pallas_call(
        _tc_mm_body,
        grid=(NBLK,),
        in_specs=[
            pl.BlockSpec((BLK, EMBED), lambda i: (i, 0)),
            pl.BlockSpec((EMBED, HID), lambda i: (0, 0)),
        ],
        out_specs=pl.BlockSpec((BLK, HID), lambda i: (i, 0)),
        out_shape=jax.ShapeDtypeStruct((NPAD, HID), jnp.float32),
    )(x, W_gcn)


def _tc_scale(xw, deg_part, b_gcn):
    return pl.pallas_call(
        _tc_scale_body,
        grid=(NBLK,),
        in_specs=[
            pl.BlockSpec((BLK, HID), lambda i: (i, 0)),
            pl.BlockSpec((BLK, 16), lambda i: (i, 0)),
            pl.BlockSpec((BLK, 16), lambda i: (i + NBLK, 0)),
            pl.BlockSpec((8, HID), lambda i: (0, 0)),
        ],
        out_specs=[
            pl.BlockSpec((BLK, HID), lambda i: (i, 0)),
            pl.BlockSpec((BLK, HID), lambda i: (i, 0)),
        ],
        out_shape=[jax.ShapeDtypeStruct((NPAD, HID), jnp.float32)] * 2,
    )(xw, deg_part, deg_part,
      jnp.broadcast_to(b_gcn[None, :], (8, HID)))


def _tc_post(t_part, w0, deg_part, wl4, bl4):
    return pl.pallas_call(
        _tc_post_body,
        grid=(NBLK,),
        in_specs=[
            pl.BlockSpec((BLK, HID), lambda i: (i, 0)),
            pl.BlockSpec((BLK, HID), lambda i: (i + NBLK, 0)),
            pl.BlockSpec((BLK, HID), lambda i: (i, 0)),
            pl.BlockSpec((BLK, 16), lambda i: (i, 0)),
            pl.BlockSpec((BLK, 16), lambda i: (i + NBLK, 0)),
            pl.BlockSpec((HID, 4), lambda i: (0, 0)),
            pl.BlockSpec((8, 4), lambda i: (0, 0)),
        ],
        out_specs=pl.BlockSpec((BLK, 4), lambda i: (i, 0)),
        out_shape=jax.ShapeDtypeStruct((NPAD, 4), jnp.float32),
    )(t_part, t_part, w0, deg_part, deg_part, wl4, bl4)


def kernel(x, edge_index, node1, node2, W_gcn, b_gcn, W_lin, b_lin):
    x = jnp.pad(x, ((0, NPAD - N_NODES), (0, 0)))
    e3 = edge_index.reshape(2, NW * NCH, CH)
    ones16 = jnp.ones((CH, 16), jnp.float32)
    zeros16 = jnp.zeros((RPT, 16), jnp.float32)
    zeros64 = jnp.zeros((128, HID), jnp.float32)

    deg_part = _deg_kernel()(e3, ones16, zeros16)
    xw = _tc_mm(x, W_gcn)
    y, w0 = _tc_scale(xw, deg_part, b_gcn)
    t_part = _scatter_kernel()(y, e3, zeros64)

    wl4 = jnp.concatenate([W_lin[:HID], W_lin[HID:]], axis=1)
    bl4 = jnp.broadcast_to(
        jnp.concatenate([b_lin, jnp.zeros((2,), jnp.float32)])[None, :], (8, 4))
    tab = _tc_post(t_part, w0, deg_part, wl4, bl4)

    return _pairs_kernel()(tab, node1, node2)
